# baseline (speedup 1.0000x reference)
import jax
import jax.numpy as jnp
from jax import lax
from jax.experimental import pallas as pl
from jax.experimental.pallas import tpu as pltpu
from jax.experimental.pallas import tpu_sc as plsc

D_MODEL = 2048
MAX_LEN = 8192

NC = 2
NS = 16
NW = NC * NS

N_TOTAL = 16384
ROWS_PER_TILE = MAX_LEN // NW
SUB = 16
NSB = ROWS_PER_TILE // SUB
SZ = N_TOTAL + 128
L = 16
VBITS = 13

_GDN = jax.lax.GatherDimensionNumbers(
    offset_dims=(), collapsed_slice_dims=(0,), start_index_map=(0,)
)


def _dyn_gather(x, idx):
    return jax.lax.gather(
        x,
        idx[:, None],
        dimension_numbers=_GDN,
        slice_sizes=(1,),
        mode=jax.lax.GatherScatterMode.PROMISE_IN_BOUNDS,
    )


def _popcount16(x):
    x = x - ((x >> 1) & 0x5555)
    x = (x & 0x3333) + ((x >> 2) & 0x3333)
    x = (x + (x >> 4)) & 0x0F0F
    return (x + (x >> 8)) & 0x1F


def _ctz(b):
    return _popcount16((b & -b) - 1)


def _pe_body(table_hbm, idx_hbm, out_hbm, idx_v, mcode, scode, arena,
             gs0, gs1, ws0, ws1):
    gsems = (gs0, gs1)
    wsems = (ws0, ws1)
    wid = lax.axis_index("s") * NC + lax.axis_index("c")
    lo = wid * ROWS_PER_TILE
    lane = lax.iota(jnp.int32, L)

    def or_tree(bv):
        y = bv
        for s in (1, 2, 4, 8):
            y = y | _dyn_gather(y, lane ^ s)
        return y

    def extract_matches(m, code, dst, base):
        bv = jnp.where(m, jnp.int32(1) << lane, jnp.int32(0))
        bits0 = or_tree(bv)[0]
        cnt = _popcount16(bits0)

        @pl.loop(0, cnt, init_carry=bits0)
        def _pop(jj, bits):
            f = _ctz(bits)
            dst[pl.ds(base + jj, L)] = _dyn_gather(
                code, jnp.full((L,), f, jnp.int32)
            )
            return bits & (bits - 1)

        return cnt

    pltpu.sync_copy(idx_hbm, idx_v)

    for b in range(2):
        pltpu.async_copy(
            table_hbm.at[pl.ds(lo + b * SUB, SUB)], arena.at[b], gsems[b]
        )

    @pl.loop(0, N_TOTAL // L, init_carry=jnp.int32(0), unroll=4)
    def _scan(j, k):
        v = idx_v[pl.ds(j * L, L)]
        m = (v >= lo) & (v < lo + ROWS_PER_TILE)
        code = ((j * L + lane) << VBITS) | v
        return k + extract_matches(m, code, mcode, k)

    k = _scan
    nvr = (k + (L - 1)) // L

    @pl.loop(0, NSB, step=2, init_carry=(jnp.int32(0), jnp.int32(0)))
    def _blocks(g, pend):
        p = list(pend)
        for b in range(2):
            sb = g + b
            sub_lo = lo + sb * SUB

            @pl.loop(0, nvr, init_carry=jnp.int32(0))
            def _subscan(j, kk):
                code = mcode[pl.ds(j * L, L)]
                vv = code & ((1 << VBITS) - 1)
                valid = (j * L + lane) < k
                mm = valid & (vv >= sub_lo) & (vv < sub_lo + SUB)
                return kk + extract_matches(mm, code, scode, kk)

            kk = _subscan

            pltpu.make_async_copy(
                table_hbm.at[pl.ds(sub_lo, SUB)], arena.at[b], gsems[b]
            ).wait()

            @pl.loop(0, (kk + (L - 1)) // L)
            def _fire(e16):
                codev = scode[pl.ds(e16 * L, L)]
                posv = codev >> VBITS
                rowv = (codev & ((1 << VBITS) - 1)) - sub_lo
                for lane_i in range(L):
                    @pl.when(e16 * L + lane_i < kk)
                    def _():
                        pltpu.async_copy(
                            arena.at[b, rowv[lane_i]],
                            out_hbm.at[posv[lane_i]],
                            wsems[b],
                        )

            total = p[b] + kk

            @pl.when(sb + 2 < NSB)
            def _():
                @pl.loop(0, total)
                def _drain(e):
                    pltpu.make_async_copy(
                        arena.at[b, 0], out_hbm.at[0], wsems[b]
                    ).wait()

                pltpu.async_copy(
                    table_hbm.at[pl.ds(sub_lo + 2 * SUB, SUB)],
                    arena.at[b],
                    gsems[b],
                )

            p[b] = jnp.where(sb + 2 < NSB, jnp.int32(0), total)
        return tuple(p)

    pend = _blocks
    for b in range(2):

        @pl.loop(0, pend[b])
        def _drain_tail(e):
            pltpu.make_async_copy(
                arena.at[b, 0], out_hbm.at[0], wsems[b]
            ).wait()


@jax.jit
def _sc_gather(table, idx):
    mesh = plsc.VectorSubcoreMesh(
        core_axis_name="c", subcore_axis_name="s", num_cores=NC, num_subcores=NS
    )
    grid_kernel = pl.kernel(
        _pe_body,
        out_type=jax.ShapeDtypeStruct((N_TOTAL, D_MODEL), jnp.float32),
        mesh=mesh,
        scratch_types=[
            pltpu.VMEM((N_TOTAL,), jnp.int32),
            pltpu.VMEM((SZ,), jnp.int32),
            pltpu.VMEM((SZ,), jnp.int32),
            pltpu.VMEM((2, SUB, D_MODEL), jnp.float32),
        ]
        + [pltpu.SemaphoreType.DMA] * 4,
    )
    return grid_kernel(table, idx)


def kernel(x, pe):
    b, l = x.shape
    table = pe.reshape(MAX_LEN, D_MODEL)
    idx = x.reshape(-1).astype(jnp.int32)
    out = _sc_gather(table, idx)
    return out.reshape(b, l, D_MODEL)


# speedup vs baseline: 1.3215x; 1.3215x over previous
import functools

import jax
import jax.numpy as jnp
from jax import lax
from jax.experimental import pallas as pl
from jax.experimental.pallas import tpu as pltpu
from jax.experimental.pallas import tpu_sc as plsc

D_MODEL = 2048
MAX_LEN = 8192

NC = 2
NS = 16
NW = NC * NS

CHUNK = 8
NBUF = 4
LOOK = 2


def _gather_body(steps, table_hbm, idx_hbm, out_hbm, idx_v, rows_v, *sems):
    gsems = sems[:NBUF]
    wsems = sems[NBUF:]
    wid = lax.axis_index("s") * NC + lax.axis_index("c")
    base = wid * (steps * CHUNK)

    def gather(t, buf):
        pltpu.async_copy(table_hbm.at[idx_v.at[t]], rows_v.at[buf], gsems[buf])

    def gather_wait(t, buf):
        pltpu.make_async_copy(
            table_hbm.at[idx_v.at[t]], rows_v.at[buf], gsems[buf]
        ).wait()

    def write(t, buf):
        pltpu.async_copy(
            rows_v.at[buf], out_hbm.at[pl.ds(base + t * CHUNK, CHUNK)], wsems[buf]
        )

    def write_wait(t, buf):
        pltpu.make_async_copy(
            rows_v.at[buf], out_hbm.at[pl.ds(base + t * CHUNK, CHUNK)], wsems[buf]
        ).wait()

    pltpu.sync_copy(idx_hbm.at[wid], idx_v)

    for b in range(LOOK):
        gather(b, b)

    @pl.loop(0, steps, step=NBUF)
    def _(g):
        for b in range(NBUF):
            s = g + b
            t = s + LOOK
            tb = (b + LOOK) % NBUF

            @pl.when(t < steps)
            def _():
                @pl.when(t >= NBUF)
                def _():
                    write_wait(t - NBUF, tb)

                gather(t, tb)

            gather_wait(s, b)
            write(s, b)

    for b in range(NBUF):
        s = steps - NBUF + b
        write_wait(s, s % NBUF)


@functools.partial(jax.jit, static_argnums=(2,))
def _sc_gather(table, idx, n):
    steps = n // (NW * CHUNK)
    mesh = plsc.VectorSubcoreMesh(
        core_axis_name="c", subcore_axis_name="s", num_cores=NC, num_subcores=NS
    )
    grid_kernel = pl.kernel(
        functools.partial(_gather_body, steps),
        out_type=jax.ShapeDtypeStruct((n, D_MODEL), jnp.float32),
        mesh=mesh,
        scratch_types=[
            pltpu.VMEM((steps, CHUNK), jnp.int32),
            pltpu.VMEM((NBUF, CHUNK, D_MODEL), jnp.float32),
        ]
        + [pltpu.SemaphoreType.DMA] * (2 * NBUF),
    )
    return grid_kernel(table, idx.reshape(NW, steps, CHUNK))


def kernel(x, pe):
    b, l = x.shape
    n = b * l
    table = pe.reshape(MAX_LEN, D_MODEL)
    idx = x.reshape(-1).astype(jnp.int32)
    out = _sc_gather(table, idx, n)
    return out.reshape(b, l, D_MODEL)
